# Initial kernel scaffold; baseline (speedup 1.0000x reference)
import jax
import jax.numpy as jnp
from jax.experimental import pallas as pl


def kernel(x, edge_index, batch_x, W1, b1, W2, b2, W3, b3):
    raise NotImplementedError("write your pallas kernel here")


# speedup vs baseline: 5.2029x; 5.2029x over previous
import functools

import jax
import jax.numpy as jnp
from jax import lax
from jax.experimental import pallas as pl
from jax.experimental.pallas import tpu as pltpu
from jax.experimental.pallas import tpu_sc as plsc

NS = 16
NC = 2
CH = 128
AUG = 16


def _sc_agg1(xcat, src2, dstp, ones16, zeros1, zerosd, *, NP, H, EPAD):
    epw = EPAD // NS
    nch = epw // CH
    rpt = NP // NS

    mesh = plsc.VectorSubcoreMesh(core_axis_name="c", subcore_axis_name="s")

    @functools.partial(
        pl.kernel,
        out_type=(jax.ShapeDtypeStruct((NP, H), jnp.float32),
                  jax.ShapeDtypeStruct((NP, H), jnp.float32),
                  jax.ShapeDtypeStruct((NP, AUG), jnp.float32)),
        mesh=mesh,
        compiler_params=pltpu.CompilerParams(use_tc_tiling_on_sc=False),
        scratch_types=[
            pltpu.VMEM((CH,), jnp.int32),
            pltpu.VMEM((CH,), jnp.int32),
            pltpu.VMEM((CH, H), jnp.float32),
            pltpu.VMEM((CH, AUG), jnp.float32),
            pltpu.VMEM_SHARED((NP, H), jnp.float32),
            pltpu.VMEM_SHARED((NP, AUG), jnp.float32),
            pltpu.SemaphoreType.DMA,
        ],
    )
    def body(xcat_h, src2_h, dstp_h, ones_h, zeros_h, zerosd_h, out0, out1,
             outd, sidx, didx, rows, onesv, acc, dacc, sem):
        c = lax.axis_index("c")
        s = lax.axis_index("s")
        r0 = s * rpt
        pltpu.sync_copy(zeros_h, acc.at[pl.ds(r0, rpt)])
        pltpu.sync_copy(zerosd_h, dacc.at[pl.ds(r0, rpt)])
        pltpu.sync_copy(ones_h, onesv)
        plsc.subcore_barrier()
        e0 = c * EPAD + s * epw
        d0 = s * epw

        def step(j, carry):
            off = j * CH
            pltpu.sync_copy(src2_h.at[pl.ds(e0 + off, CH)], sidx)
            pltpu.sync_copy(dstp_h.at[pl.ds(d0 + off, CH)], didx)
            pltpu.async_copy(xcat_h.at[sidx], rows, sem).wait()
            pltpu.sync_copy(rows, acc.at[didx], add=True)

            @pl.when(c == 0)
            def _():
                pltpu.sync_copy(onesv, dacc.at[didx], add=True)

            return carry

        lax.fori_loop(0, nch, step, 0)
        plsc.subcore_barrier()

        @pl.when(c == 0)
        def _():
            pltpu.sync_copy(acc.at[pl.ds(r0, rpt)], out0.at[pl.ds(r0, rpt)])
            pltpu.sync_copy(dacc.at[pl.ds(r0, rpt)], outd.at[pl.ds(r0, rpt)])

        @pl.when(c == 1)
        def _():
            pltpu.sync_copy(acc.at[pl.ds(r0, rpt)], out1.at[pl.ds(r0, rpt)])

    return body(xcat, src2, dstp, ones16, zeros1, zerosd)


def _sc_agg2(q, srcp, dstp, zeros2, *, NP, EPAD):
    epw = EPAD // (NC * NS)
    nch = epw // CH
    rpt = NP // NS

    mesh = plsc.VectorSubcoreMesh(core_axis_name="c", subcore_axis_name="s")

    @functools.partial(
        pl.kernel,
        out_type=(jax.ShapeDtypeStruct((NP, AUG), jnp.float32),
                  jax.ShapeDtypeStruct((NP, AUG), jnp.float32)),
        mesh=mesh,
        compiler_params=pltpu.CompilerParams(use_tc_tiling_on_sc=False),
        scratch_types=[
            pltpu.VMEM((CH,), jnp.int32),
            pltpu.VMEM((CH,), jnp.int32),
            pltpu.VMEM((CH, AUG), jnp.float32),
            pltpu.VMEM_SHARED((NP, AUG), jnp.float32),
            pltpu.SemaphoreType.DMA,
        ],
    )
    def body(q_h, srcp_h, dstp_h, zeros_h, out0, out1, sidx, didx, rows, acc,
             sem):
        c = lax.axis_index("c")
        s = lax.axis_index("s")
        r0 = s * rpt
        pltpu.sync_copy(zeros_h, acc.at[pl.ds(r0, rpt)])
        plsc.subcore_barrier()
        e0 = (c * NS + s) * epw

        def step(j, carry):
            off = j * CH
            pltpu.sync_copy(srcp_h.at[pl.ds(e0 + off, CH)], sidx)
            pltpu.sync_copy(dstp_h.at[pl.ds(e0 + off, CH)], didx)
            pltpu.async_copy(q_h.at[sidx], rows, sem).wait()
            pltpu.sync_copy(rows, acc.at[didx], add=True)
            return carry

        lax.fori_loop(0, nch, step, 0)
        plsc.subcore_barrier()

        @pl.when(c == 0)
        def _():
            pltpu.sync_copy(acc.at[pl.ds(r0, rpt)], out0.at[pl.ds(r0, rpt)])

        @pl.when(c == 1)
        def _():
            pltpu.sync_copy(acc.at[pl.ds(r0, rpt)], out1.at[pl.ds(r0, rpt)])

    return body(q, srcp, dstp, zeros2)


def _tc1(x, alo, ahi, degp, A1, C1lo, C1hi, b1r, A2p, b2p, C2p, *, N, F, H, BN):
    grid = (N // BN,)

    def body(x_r, alo_r, ahi_r, deg_r, a1_r, c1lo_r, c1hi_r, b1_r, a2_r, b2_r,
             c2_r, q_r, w_r):
        deg = deg_r[...][:, 0:1]
        pre = jnp.dot(x_r[...], a1_r[...],
                      preferred_element_type=jnp.float32) + b1_r[...]
        aggc = (jnp.dot(alo_r[...], c1lo_r[...],
                        preferred_element_type=jnp.float32)
                + jnp.dot(ahi_r[...], c1hi_r[...],
                          preferred_element_type=jnp.float32))
        h1 = jnp.maximum(deg * pre + aggc, 0.0)
        q_r[...] = jnp.dot(h1, c2_r[...], preferred_element_type=jnp.float32)
        w_r[...] = deg * (jnp.dot(h1, a2_r[...],
                                  preferred_element_type=jnp.float32)
                          + b2_r[...])

    return pl.pallas_call(
        body,
        grid=grid,
        in_specs=[
            pl.BlockSpec((BN, F), lambda i: (i, 0)),
            pl.BlockSpec((BN, H), lambda i: (i, 0)),
            pl.BlockSpec((BN, H), lambda i: (i, 0)),
            pl.BlockSpec((BN, AUG), lambda i: (i, 0)),
            pl.BlockSpec((F, F), lambda i: (0, 0)),
            pl.BlockSpec((H, F), lambda i: (0, 0)),
            pl.BlockSpec((H, F), lambda i: (0, 0)),
            pl.BlockSpec((1, F), lambda i: (0, 0)),
            pl.BlockSpec((F, AUG), lambda i: (0, 0)),
            pl.BlockSpec((1, AUG), lambda i: (0, 0)),
            pl.BlockSpec((F, AUG), lambda i: (0, 0)),
        ],
        out_specs=[
            pl.BlockSpec((BN, AUG), lambda i: (i, 0)),
            pl.BlockSpec((BN, AUG), lambda i: (i, 0)),
        ],
        out_shape=[
            jax.ShapeDtypeStruct((N, AUG), jnp.float32),
            jax.ShapeDtypeStruct((N, AUG), jnp.float32),
        ],
    )(x, alo, ahi, degp, A1, C1lo, C1hi, b1r, A2p, b2p, C2p)


def _tc2(w, a2a, a2b, bx2, W3r, b3r, *, N, B, BN):
    grid = (N // BN,)

    def body(w_r, a_r, b_r, bx_r, w3_r, b3_r, out_r):
        i = pl.program_id(0)
        h2 = jnp.maximum(w_r[...] + a_r[...] + b_r[...], 0.0)
        h3 = jnp.sum(h2 * w3_r[...], axis=1, keepdims=True) + b3_r[0, 0]
        seg = lax.broadcasted_iota(jnp.int32, (BN, B), 1)
        onehot = (bx_r[...] == seg).astype(jnp.float32)
        part = jnp.sum(onehot * h3, axis=0, keepdims=True)

        @pl.when(i == 0)
        def _():
            out_r[...] = jnp.zeros_like(out_r)

        out_r[...] += part

    return pl.pallas_call(
        body,
        grid=grid,
        in_specs=[
            pl.BlockSpec((BN, AUG), lambda i: (i, 0)),
            pl.BlockSpec((BN, AUG), lambda i: (i, 0)),
            pl.BlockSpec((BN, AUG), lambda i: (i, 0)),
            pl.BlockSpec((BN, 1), lambda i: (i, 0)),
            pl.BlockSpec((1, AUG), lambda i: (0, 0)),
            pl.BlockSpec((1, 1), lambda i: (0, 0)),
        ],
        out_specs=pl.BlockSpec((1, B), lambda i: (0, 0)),
        out_shape=jax.ShapeDtypeStruct((1, B), jnp.float32),
    )(w, a2a, a2b, bx2, W3r, b3r)


def kernel(x, edge_index, batch_x, W1, b1, W2, b2, W3, b3):
    N, F = x.shape
    E = edge_index.shape[1]
    B = 64
    H = F // 2
    BN = 1000
    NP = ((N + 1 + NS * 8 - 1) // (NS * 8)) * (NS * 8)

    grp = NC * NS * CH
    EPAD = ((E + grp - 1) // grp) * grp
    npad = EPAD - E

    src = edge_index[0]
    dst = edge_index[1]
    src_p = jnp.concatenate([src, jnp.zeros((npad,), jnp.int32)])
    dst_p = jnp.concatenate([dst, jnp.full((npad,), N, jnp.int32)])
    src2 = jnp.concatenate([src_p, src_p + N])

    xcat = jnp.concatenate([x[:, :H], x[:, H:]], axis=0)

    zeros1 = jnp.zeros((NP // NS, H), jnp.float32)
    zerosd = jnp.zeros((NP // NS, AUG), jnp.float32)
    ones16 = jnp.ones((CH, AUG), jnp.float32)

    A1 = (W1[:, :F] - W1[:, F:]).T
    C1 = W1[:, F:].T
    C1lo, C1hi = C1[:H], C1[H:]
    b1r = b1.reshape(1, F)
    K2 = W2.shape[0]
    A2p = jnp.zeros((F, AUG), jnp.float32).at[:, :K2].set((W2[:, :F] - W2[:, F:]).T)
    C2p = jnp.zeros((F, AUG), jnp.float32).at[:, :K2].set(W2[:, F:].T)
    b2p = jnp.zeros((1, AUG), jnp.float32).at[0, :K2].set(b2)
    W3r = jnp.zeros((1, AUG), jnp.float32).at[0, :K2].set(W3[0])
    b3r = b3.reshape(1, 1)
    bx2 = batch_x.reshape(N, 1)

    alo, ahi, degp = _sc_agg1(xcat, src2, dst_p, ones16, zeros1, zerosd,
                              NP=NP, H=H, EPAD=EPAD)
    q, w = _tc1(x, alo[:N], ahi[:N], degp[:N], A1, C1lo, C1hi, b1r, A2p, b2p,
                C2p, N=N, F=F, H=H, BN=BN)
    a2a, a2b = _sc_agg2(q, src_p, dst_p, zerosd, NP=NP, EPAD=EPAD)
    energy = _tc2(w, a2a[:N], a2b[:N], bx2, W3r, b3r, N=N, B=B, BN=BN)
    return energy.reshape(B, 1)


# speedup vs baseline: 7.1774x; 1.3795x over previous
import functools

import jax
import jax.numpy as jnp
from jax import lax
from jax.experimental import pallas as pl
from jax.experimental.pallas import tpu as pltpu
from jax.experimental.pallas import tpu_sc as plsc

NS = 16
NC = 2
CH = 64
AUG = 16


def _sc_agg1(xcat, src2r, dstpr, ones16, zeros1, zerosd, *, NP, H, EPAD):
    epw = EPAD // NS
    nch = epw // CH
    rpt = NP // NS

    mesh = plsc.VectorSubcoreMesh(core_axis_name="c", subcore_axis_name="s")

    @functools.partial(
        pl.kernel,
        out_type=(jax.ShapeDtypeStruct((NP, H), jnp.float32),
                  jax.ShapeDtypeStruct((NP, H), jnp.float32),
                  jax.ShapeDtypeStruct((NP, AUG), jnp.float32)),
        mesh=mesh,
        compiler_params=pltpu.CompilerParams(use_tc_tiling_on_sc=False),
        scratch_types=[
            pltpu.VMEM((nch, CH), jnp.int32),
            pltpu.VMEM((nch, CH), jnp.int32),
            pltpu.VMEM((CH, H), jnp.float32),
            pltpu.VMEM((CH, H), jnp.float32),
            pltpu.VMEM((CH, AUG), jnp.float32),
            pltpu.VMEM_SHARED((NP, H), jnp.float32),
            pltpu.VMEM_SHARED((NP, AUG), jnp.float32),
            pltpu.SemaphoreType.DMA,
            pltpu.SemaphoreType.DMA,
            pltpu.SemaphoreType.DMA,
            pltpu.SemaphoreType.DMA,
            pltpu.SemaphoreType.DMA,
        ],
    )
    def body(xcat_h, src2_h, dstp_h, ones_h, zeros_h, zerosd_h, out0, out1,
             outd, sidx, didx, rows0, rows1, onesv, acc, dacc, semg0, semg1,
             sems0, sems1, semd):
        c = lax.axis_index("c")
        s = lax.axis_index("s")
        r0 = s * rpt
        pltpu.sync_copy(zeros_h, acc.at[pl.ds(r0, rpt)])
        pltpu.sync_copy(zerosd_h, dacc.at[pl.ds(r0, rpt)])
        pltpu.sync_copy(ones_h, onesv)
        pltpu.sync_copy(src2_h.at[pl.ds(c * (EPAD // CH) + s * nch, nch)], sidx)
        pltpu.sync_copy(dstp_h.at[pl.ds(s * nch, nch)], didx)
        plsc.subcore_barrier()

        bufs = (rows0, rows1)
        semg = (semg0, semg1)
        sems = (sems0, sems1)
        pltpu.async_copy(xcat_h.at[sidx.at[0]], rows0, semg0)

        def step(g, carry):
            for b in range(2):
                j = 2 * g + b
                o = 1 - b

                @pl.when(j + 1 < nch)
                def _():
                    @pl.when(j >= 1)
                    def _():
                        pltpu.make_async_copy(
                            bufs[o], acc.at[didx.at[j]], sems[o]).wait()

                    pltpu.async_copy(xcat_h.at[sidx.at[j + 1]], bufs[o],
                                     semg[o])

                pltpu.make_async_copy(xcat_h.at[sidx.at[j]], bufs[b],
                                      semg[b]).wait()
                pltpu.async_copy(bufs[b], acc.at[didx.at[j]], sems[b],
                                 add=True)

                @pl.when(c == 0)
                def _():
                    @pl.when(j >= 1)
                    def _():
                        pltpu.make_async_copy(
                            onesv, dacc.at[didx.at[j]], semd).wait()

                    pltpu.async_copy(onesv, dacc.at[didx.at[j]], semd,
                                     add=True)

            return carry

        lax.fori_loop(0, nch // 2, step, 0)
        pltpu.make_async_copy(bufs[0], acc.at[didx.at[0]], sems[0]).wait()
        pltpu.make_async_copy(bufs[1], acc.at[didx.at[0]], sems[1]).wait()

        @pl.when(c == 0)
        def _():
            pltpu.make_async_copy(onesv, dacc.at[didx.at[0]], semd).wait()

        plsc.subcore_barrier()

        @pl.when(c == 0)
        def _():
            pltpu.sync_copy(acc.at[pl.ds(r0, rpt)], out0.at[pl.ds(r0, rpt)])
            pltpu.sync_copy(dacc.at[pl.ds(r0, rpt)], outd.at[pl.ds(r0, rpt)])

        @pl.when(c == 1)
        def _():
            pltpu.sync_copy(acc.at[pl.ds(r0, rpt)], out1.at[pl.ds(r0, rpt)])

    return body(xcat, src2r, dstpr, ones16, zeros1, zerosd)


def _sc_agg2(q, srcpr, dstpr, zeros2, *, NP, EPAD):
    epw = EPAD // (NC * NS)
    nch = epw // CH
    rpt = NP // NS

    mesh = plsc.VectorSubcoreMesh(core_axis_name="c", subcore_axis_name="s")

    @functools.partial(
        pl.kernel,
        out_type=(jax.ShapeDtypeStruct((NP, AUG), jnp.float32),
                  jax.ShapeDtypeStruct((NP, AUG), jnp.float32)),
        mesh=mesh,
        compiler_params=pltpu.CompilerParams(use_tc_tiling_on_sc=False),
        scratch_types=[
            pltpu.VMEM((nch, CH), jnp.int32),
            pltpu.VMEM((nch, CH), jnp.int32),
            pltpu.VMEM((CH, AUG), jnp.float32),
            pltpu.VMEM((CH, AUG), jnp.float32),
            pltpu.VMEM_SHARED((NP, AUG), jnp.float32),
            pltpu.SemaphoreType.DMA,
            pltpu.SemaphoreType.DMA,
            pltpu.SemaphoreType.DMA,
            pltpu.SemaphoreType.DMA,
        ],
    )
    def body(q_h, srcp_h, dstp_h, zeros_h, out0, out1, sidx, didx, rows0,
             rows1, acc, semg0, semg1, sems0, sems1):
        c = lax.axis_index("c")
        s = lax.axis_index("s")
        r0 = s * rpt
        w = c * NS + s
        pltpu.sync_copy(zeros_h, acc.at[pl.ds(r0, rpt)])
        pltpu.sync_copy(srcp_h.at[pl.ds(w * nch, nch)], sidx)
        pltpu.sync_copy(dstp_h.at[pl.ds(w * nch, nch)], didx)
        plsc.subcore_barrier()

        bufs = (rows0, rows1)
        semg = (semg0, semg1)
        sems = (sems0, sems1)
        pltpu.async_copy(q_h.at[sidx.at[0]], rows0, semg0)

        def step(g, carry):
            for b in range(2):
                j = 2 * g + b
                o = 1 - b

                @pl.when(j + 1 < nch)
                def _():
                    @pl.when(j >= 1)
                    def _():
                        pltpu.make_async_copy(
                            bufs[o], acc.at[didx.at[j]], sems[o]).wait()

                    pltpu.async_copy(q_h.at[sidx.at[j + 1]], bufs[o], semg[o])

                pltpu.make_async_copy(q_h.at[sidx.at[j]], bufs[b],
                                      semg[b]).wait()
                pltpu.async_copy(bufs[b], acc.at[didx.at[j]], sems[b],
                                 add=True)

            return carry

        lax.fori_loop(0, nch // 2, step, 0)
        pltpu.make_async_copy(bufs[0], acc.at[didx.at[0]], sems[0]).wait()
        pltpu.make_async_copy(bufs[1], acc.at[didx.at[0]], sems[1]).wait()
        plsc.subcore_barrier()

        @pl.when(c == 0)
        def _():
            pltpu.sync_copy(acc.at[pl.ds(r0, rpt)], out0.at[pl.ds(r0, rpt)])

        @pl.when(c == 1)
        def _():
            pltpu.sync_copy(acc.at[pl.ds(r0, rpt)], out1.at[pl.ds(r0, rpt)])

    return body(q, srcpr, dstpr, zeros2)


def _tc1(x, alo, ahi, degp, A1, C1lo, C1hi, b1r, A2p, b2p, C2p, *, N, F, H, BN):
    grid = (N // BN,)

    def body(x_r, alo_r, ahi_r, deg_r, a1_r, c1lo_r, c1hi_r, b1_r, a2_r, b2_r,
             c2_r, q_r, w_r):
        deg = deg_r[...][:, 0:1]
        pre = jnp.dot(x_r[...], a1_r[...],
                      preferred_element_type=jnp.float32) + b1_r[...]
        aggc = (jnp.dot(alo_r[...], c1lo_r[...],
                        preferred_element_type=jnp.float32)
                + jnp.dot(ahi_r[...], c1hi_r[...],
                          preferred_element_type=jnp.float32))
        h1 = jnp.maximum(deg * pre + aggc, 0.0)
        q_r[...] = jnp.dot(h1, c2_r[...], preferred_element_type=jnp.float32)
        w_r[...] = deg * (jnp.dot(h1, a2_r[...],
                                  preferred_element_type=jnp.float32)
                          + b2_r[...])

    return pl.pallas_call(
        body,
        grid=grid,
        in_specs=[
            pl.BlockSpec((BN, F), lambda i: (i, 0)),
            pl.BlockSpec((BN, H), lambda i: (i, 0)),
            pl.BlockSpec((BN, H), lambda i: (i, 0)),
            pl.BlockSpec((BN, AUG), lambda i: (i, 0)),
            pl.BlockSpec((F, F), lambda i: (0, 0)),
            pl.BlockSpec((H, F), lambda i: (0, 0)),
            pl.BlockSpec((H, F), lambda i: (0, 0)),
            pl.BlockSpec((1, F), lambda i: (0, 0)),
            pl.BlockSpec((F, AUG), lambda i: (0, 0)),
            pl.BlockSpec((1, AUG), lambda i: (0, 0)),
            pl.BlockSpec((F, AUG), lambda i: (0, 0)),
        ],
        out_specs=[
            pl.BlockSpec((BN, AUG), lambda i: (i, 0)),
            pl.BlockSpec((BN, AUG), lambda i: (i, 0)),
        ],
        out_shape=[
            jax.ShapeDtypeStruct((N, AUG), jnp.float32),
            jax.ShapeDtypeStruct((N, AUG), jnp.float32),
        ],
    )(x, alo, ahi, degp, A1, C1lo, C1hi, b1r, A2p, b2p, C2p)


def _tc2(w, a2a, a2b, bx2, W3r, b3r, *, N, B, BN):
    grid = (N // BN,)

    def body(w_r, a_r, b_r, bx_r, w3_r, b3_r, out_r):
        i = pl.program_id(0)
        h2 = jnp.maximum(w_r[...] + a_r[...] + b_r[...], 0.0)
        h3 = jnp.sum(h2 * w3_r[...], axis=1, keepdims=True) + b3_r[0, 0]
        seg = lax.broadcasted_iota(jnp.int32, (BN, B), 1)
        onehot = (bx_r[...] == seg).astype(jnp.float32)
        part = jnp.sum(onehot * h3, axis=0, keepdims=True)

        @pl.when(i == 0)
        def _():
            out_r[...] = jnp.zeros_like(out_r)

        out_r[...] += part

    return pl.pallas_call(
        body,
        grid=grid,
        in_specs=[
            pl.BlockSpec((BN, AUG), lambda i: (i, 0)),
            pl.BlockSpec((BN, AUG), lambda i: (i, 0)),
            pl.BlockSpec((BN, AUG), lambda i: (i, 0)),
            pl.BlockSpec((BN, 1), lambda i: (i, 0)),
            pl.BlockSpec((1, AUG), lambda i: (0, 0)),
            pl.BlockSpec((1, 1), lambda i: (0, 0)),
        ],
        out_specs=pl.BlockSpec((1, B), lambda i: (0, 0)),
        out_shape=jax.ShapeDtypeStruct((1, B), jnp.float32),
    )(w, a2a, a2b, bx2, W3r, b3r)


def kernel(x, edge_index, batch_x, W1, b1, W2, b2, W3, b3):
    N, F = x.shape
    E = edge_index.shape[1]
    B = 64
    H = F // 2
    BN = 1000
    NP = ((N + 1 + NS * 8 - 1) // (NS * 8)) * (NS * 8)

    grp = NC * NS * CH * 2
    EPAD = ((E + grp - 1) // grp) * grp
    npad = EPAD - E

    src = edge_index[0]
    dst = edge_index[1]
    src_p = jnp.concatenate([src, jnp.zeros((npad,), jnp.int32)])
    dst_p = jnp.concatenate([dst, jnp.full((npad,), N, jnp.int32)])
    src2 = jnp.concatenate([src_p, src_p + N])

    xcat = jnp.concatenate([x[:, :H], x[:, H:]], axis=0)

    zeros1 = jnp.zeros((NP // NS, H), jnp.float32)
    zerosd = jnp.zeros((NP // NS, AUG), jnp.float32)
    ones16 = jnp.ones((CH, AUG), jnp.float32)

    A1 = (W1[:, :F] - W1[:, F:]).T
    C1 = W1[:, F:].T
    C1lo, C1hi = C1[:H], C1[H:]
    b1r = b1.reshape(1, F)
    K2 = W2.shape[0]
    A2p = jnp.zeros((F, AUG), jnp.float32).at[:, :K2].set((W2[:, :F] - W2[:, F:]).T)
    C2p = jnp.zeros((F, AUG), jnp.float32).at[:, :K2].set(W2[:, F:].T)
    b2p = jnp.zeros((1, AUG), jnp.float32).at[0, :K2].set(b2)
    W3r = jnp.zeros((1, AUG), jnp.float32).at[0, :K2].set(W3[0])
    b3r = b3.reshape(1, 1)
    bx2 = batch_x.reshape(N, 1)

    src2r = src2.reshape(2 * EPAD // CH, CH)
    dstpr = dst_p.reshape(EPAD // CH, CH)
    alo, ahi, degp = _sc_agg1(xcat, src2r, dstpr, ones16, zeros1, zerosd,
                              NP=NP, H=H, EPAD=EPAD)
    q, w = _tc1(x, alo[:N], ahi[:N], degp[:N], A1, C1lo, C1hi, b1r, A2p, b2p,
                C2p, N=N, F=F, H=H, BN=BN)
    srcpr = src2r[:EPAD // CH]
    a2a, a2b = _sc_agg2(q, srcpr, dstpr, zerosd, NP=NP, EPAD=EPAD)
    energy = _tc2(w, a2a[:N], a2b[:N], bx2, W3r, b3r, N=N, B=B, BN=BN)
    return energy.reshape(B, 1)
